# baseline (speedup 1.0000x reference)
import functools

import jax
import jax.numpy as jnp
from jax import lax
from jax.experimental import pallas as pl
from jax.experimental.pallas import tpu as pltpu
from jax.experimental.pallas import tpu_sc as plsc

N_NODES = 10000
D = 128
N_EDGES = 320000

NC = 2
NS = 16
LANES = 16
NW = NC * NS
EPW = N_EDGES // NW
CHUNK = 2000
NCHUNK = EPW // CHUNK
CPAD = 2176

BM = 10000


def _tc_body(x_ref, w1_ref, b1_ref, w2_ref, b2_ref, a_ref, out_ref):
    xb = x_ref[...]
    h1 = lax.dot_general(xb, w1_ref[...], (((1,), (1,)), ((), ())),
                         preferred_element_type=jnp.float32)
    h1 = jnp.maximum(h1 + b1_ref[...], 0.0)
    h2 = lax.dot_general(h1, w2_ref[...], (((1,), (1,)), ((), ())),
                         preferred_element_type=jnp.float32)
    h2 = h2 + b2_ref[...]
    s = lax.dot_general(a_ref[...], h2, (((1,), (1,)), ((), ())),
                        preferred_element_type=jnp.float32)
    u = lax.bitcast_convert_type(s, jnp.uint32)
    r = (u + 0x7FFF + ((u >> 16) & 1)) >> 16
    packed = (r[0:1, :] << 16) | r[1:2, :]
    out_ref[...] = lax.bitcast_convert_type(packed, jnp.int32)


def _node_scores(x, W1, b1, W2, b2, a_resh):
    return pl.pallas_call(
        _tc_body,
        grid=(pl.cdiv(N_NODES, BM),),
        in_specs=[
            pl.BlockSpec((BM, D), lambda i: (i, 0)),
            pl.BlockSpec((D, D), lambda i: (0, 0)),
            pl.BlockSpec((1, D), lambda i: (0, 0)),
            pl.BlockSpec((D, D), lambda i: (0, 0)),
            pl.BlockSpec((1, D), lambda i: (0, 0)),
            pl.BlockSpec((2, D), lambda i: (0, 0)),
        ],
        out_specs=pl.BlockSpec((1, BM), lambda i: (0, i)),
        out_shape=jax.ShapeDtypeStruct((1, N_NODES), jnp.int32),
    )(x, W1, b1, W2, b2, a_resh)


_sc_mesh = plsc.VectorSubcoreMesh(core_axis_name="c", subcore_axis_name="s")


@functools.partial(
    pl.kernel,
    mesh=_sc_mesh,
    compiler_params=pltpu.CompilerParams(needs_layout_passes=False),
    out_type=jax.ShapeDtypeStruct((N_EDGES,), jnp.float32),
    scratch_types=[
        pltpu.VMEM((1, N_NODES), jnp.int32),
        pltpu.VMEM((2, CPAD), jnp.int32),
        pltpu.VMEM((2, CPAD), jnp.int32),
        pltpu.VMEM((CHUNK,), jnp.float32),
        pltpu.VMEM((CHUNK,), jnp.float32),
        pltpu.SemaphoreType.DMA,
        pltpu.SemaphoreType.DMA,
        pltpu.SemaphoreType.DMA,
    ],
)
def _sc_edge(tab_hbm, edge_hbm, out_hbm, tab_v, sd_a, sd_b, res_a, res_b,
             semt, semi, semo):
    sd_bufs = (sd_a, sd_b)
    res_bufs = (res_a, res_b)
    wid = lax.axis_index("s") * NC + lax.axis_index("c")
    base = wid * EPW

    def chunk_in(k, buf):
        bk = base + k * CHUNK
        ab = jnp.minimum((bk // 128) * 128, N_EDGES - CPAD)
        return pltpu.async_copy(edge_hbm.at[:, pl.ds(ab, CPAD)],
                                sd_bufs[buf], semi), bk - ab

    ct = pltpu.async_copy(tab_hbm, tab_v, semt)
    ins = {}
    ins[0] = chunk_in(0, 0)
    ins[1] = chunk_in(1, 1)
    ct.wait()

    zero16 = jnp.zeros((LANES,), jnp.int32)
    hi_mask = jnp.full((LANES,), -65536, jnp.int32)
    outs = {}
    for k in range(NCHUNK):
        buf = k % 2
        sd_k = sd_bufs[buf]
        res_k = res_bufs[buf]
        cin, off = ins[k]
        cin.wait()
        if k >= 2:
            outs[k - 2].wait()

        @plsc.parallel_loop(0, CHUNK, step=LANES, unroll=5)
        def body(i):
            sl_in = pl.ds(off + i, LANES)
            w_s = plsc.load_gather(tab_v, [zero16, sd_k[0, sl_in]])
            w_d = plsc.load_gather(tab_v, [zero16, sd_k[1, sl_in]])
            s1 = plsc.bitcast(w_s & hi_mask, jnp.float32)
            s2 = plsc.bitcast(w_d << 16, jnp.float32)
            v = s1 + s2
            t = jnp.exp(-2.0 * jnp.abs(v))
            r = (1.0 - t) / (1.0 + t)
            res_k[pl.ds(i, LANES)] = jnp.where(v < 0.0, -r, r)

        outs[k] = pltpu.async_copy(
            res_k, out_hbm.at[pl.ds(base + k * CHUNK, CHUNK)], semo)
        if k + 2 < NCHUNK:
            ins[k + 2] = chunk_in(k + 2, buf)
    outs[NCHUNK - 2].wait()
    outs[NCHUNK - 1].wait()


def kernel(x, edge_index, W1, b1, W2, b2, a):
    s = _node_scores(x, W1, b1.reshape(1, D), W2, b2.reshape(1, D),
                     a.reshape(2, D))
    return _sc_edge(s, edge_index)


# speedup vs baseline: 1.0069x; 1.0069x over previous
import functools

import jax
import jax.numpy as jnp
from jax import lax
from jax.experimental import pallas as pl
from jax.experimental.pallas import tpu as pltpu
from jax.experimental.pallas import tpu_sc as plsc

N_NODES = 10000
D = 128
N_EDGES = 320000

NC = 2
NS = 16
LANES = 16
NW = NC * NS
EPW = N_EDGES // NW
CHUNK = 2000
NCHUNK = EPW // CHUNK
CPAD = 2176

BM = 5120


def _tc_body(x_ref, w1_ref, b1_ref, w2_ref, b2_ref, a_ref, out_ref):
    xb = x_ref[...]
    h1 = lax.dot_general(xb, w1_ref[...], (((1,), (1,)), ((), ())),
                         preferred_element_type=jnp.float32)
    h1 = jnp.maximum(h1 + b1_ref[...], 0.0)
    h2 = lax.dot_general(h1, w2_ref[...], (((1,), (1,)), ((), ())),
                         preferred_element_type=jnp.float32)
    h2 = h2 + b2_ref[...]
    s = lax.dot_general(a_ref[...], h2, (((1,), (1,)), ((), ())),
                        preferred_element_type=jnp.float32)
    u = lax.bitcast_convert_type(s, jnp.uint32)
    r = (u + 0x7FFF + ((u >> 16) & 1)) >> 16
    packed = (r[0:1, :] << 16) | r[1:2, :]
    out_ref[...] = lax.bitcast_convert_type(packed, jnp.int32)


def _node_scores(x, W1, b1, W2, b2, a_resh):
    return pl.pallas_call(
        _tc_body,
        grid=(pl.cdiv(N_NODES, BM),),
        in_specs=[
            pl.BlockSpec((BM, D), lambda i: (i, 0)),
            pl.BlockSpec((D, D), lambda i: (0, 0)),
            pl.BlockSpec((1, D), lambda i: (0, 0)),
            pl.BlockSpec((D, D), lambda i: (0, 0)),
            pl.BlockSpec((1, D), lambda i: (0, 0)),
            pl.BlockSpec((2, D), lambda i: (0, 0)),
        ],
        out_specs=pl.BlockSpec((1, BM), lambda i: (0, i)),
        out_shape=jax.ShapeDtypeStruct((1, N_NODES), jnp.int32),
    )(x, W1, b1, W2, b2, a_resh)


_sc_mesh = plsc.VectorSubcoreMesh(core_axis_name="c", subcore_axis_name="s")


@functools.partial(
    pl.kernel,
    mesh=_sc_mesh,
    compiler_params=pltpu.CompilerParams(needs_layout_passes=False),
    out_type=jax.ShapeDtypeStruct((N_EDGES,), jnp.float32),
    scratch_types=[
        pltpu.VMEM((1, N_NODES), jnp.int32),
        pltpu.VMEM((2, CPAD), jnp.int32),
        pltpu.VMEM((2, CPAD), jnp.int32),
        pltpu.VMEM((CHUNK,), jnp.float32),
        pltpu.VMEM((CHUNK,), jnp.float32),
        pltpu.SemaphoreType.DMA,
        pltpu.SemaphoreType.DMA,
        pltpu.SemaphoreType.DMA,
    ],
)
def _sc_edge(tab_hbm, edge_hbm, out_hbm, tab_v, sd_a, sd_b, res_a, res_b,
             semt, semi, semo):
    sd_bufs = (sd_a, sd_b)
    res_bufs = (res_a, res_b)
    wid = lax.axis_index("s") * NC + lax.axis_index("c")
    base = wid * EPW

    def chunk_in(k, buf):
        bk = base + k * CHUNK
        ab = jnp.minimum((bk // 128) * 128, N_EDGES - CPAD)
        return pltpu.async_copy(edge_hbm.at[:, pl.ds(ab, CPAD)],
                                sd_bufs[buf], semi), bk - ab

    ct = pltpu.async_copy(tab_hbm, tab_v, semt)
    ins = {}
    ins[0] = chunk_in(0, 0)
    ins[1] = chunk_in(1, 1)
    ct.wait()

    zero16 = jnp.zeros((LANES,), jnp.int32)
    hi_mask = jnp.full((LANES,), -65536, jnp.int32)
    outs = {}
    for k in range(NCHUNK):
        buf = k % 2
        sd_k = sd_bufs[buf]
        res_k = res_bufs[buf]
        cin, off = ins[k]
        cin.wait()
        if k >= 2:
            outs[k - 2].wait()

        @plsc.parallel_loop(0, CHUNK, step=LANES, unroll=5)
        def body(i):
            sl_in = pl.ds(off + i, LANES)
            w_s = plsc.load_gather(tab_v, [zero16, sd_k[0, sl_in]])
            w_d = plsc.load_gather(tab_v, [zero16, sd_k[1, sl_in]])
            s1 = plsc.bitcast(w_s & hi_mask, jnp.float32)
            s2 = plsc.bitcast(w_d << 16, jnp.float32)
            v = s1 + s2
            t = jnp.exp(-2.0 * jnp.abs(v))
            r = (1.0 - t) / (1.0 + t)
            res_k[pl.ds(i, LANES)] = jnp.where(v < 0.0, -r, r)

        outs[k] = pltpu.async_copy(
            res_k, out_hbm.at[pl.ds(base + k * CHUNK, CHUNK)], semo)
        if k + 2 < NCHUNK:
            ins[k + 2] = chunk_in(k + 2, buf)
    outs[NCHUNK - 2].wait()
    outs[NCHUNK - 1].wait()


def kernel(x, edge_index, W1, b1, W2, b2, a):
    s = _node_scores(x, W1, b1.reshape(1, D), W2, b2.reshape(1, D),
                     a.reshape(2, D))
    return _sc_edge(s, edge_index)
